# baseline (speedup 1.0000x reference)
import functools

import jax
import jax.numpy as jnp
from jax import lax
from jax.experimental import pallas as pl
from jax.experimental.pallas import tpu as pltpu
from jax.experimental.pallas import tpu_sc as plsc

_NU = 25000
_NTOT = 50000
_D = 64
_NE = 800000
_B = 4096
_NSLOT = 2 * _B
_L = 16
_NS = 16
_NC = 2
_NW = _NC * _NS

_NT128 = _NE // 128
_T_LO = _NT128 // _NW
_T_HI = _T_LO + 1
_N_HI = _NT128 - _NW * _T_LO
_VEC_HI = _T_HI * 8
_VEC_LO = _T_LO * 8
_CHUNK_V = 128
_CHUNK_E = _CHUNK_V * _L
_N_CHUNKS = (_VEC_HI + _CHUNK_V - 1) // _CHUNK_V
_CMAX = ((_VEC_HI * _L + 63) // 64) * 64
_BLK = 64
_DUMMY = _NSLOT
_ACC_ROWS = _NSLOT + 128
_ZROWS = _ACC_ROWS // _NS
_JPT = _NSLOT // _NS
_NCH = 512


def _make_sc_kernel():
    mesh = plsc.VectorSubcoreMesh(core_axis_name="c", subcore_axis_name="s")

    @functools.partial(
        pl.kernel,
        out_type=jax.ShapeDtypeStruct((_NC, _NSLOT, _D), jnp.float32),
        mesh=mesh,
        scratch_types=[
            pltpu.VMEM((_NTOT,), jnp.int32),
            pltpu.VMEM((_NCH,), jnp.int32),
            pltpu.VMEM((2, _CHUNK_E), jnp.int32),
            pltpu.VMEM((2, _CHUNK_E), jnp.int32),
            pltpu.VMEM((_CMAX,), jnp.int32),
            pltpu.VMEM((2 * _BLK,), jnp.int32),
            pltpu.VMEM((_BLK,), jnp.int32),
            pltpu.VMEM((_BLK,), jnp.int32),
            pltpu.VMEM((2 * _BLK, _D), jnp.bfloat16),
            pltpu.VMEM((2 * _BLK, _D), jnp.float32),
            pltpu.VMEM((_JPT,), jnp.int32),
            pltpu.VMEM((_L,), jnp.int32),
            pltpu.VMEM_SHARED((_ACC_ROWS, _D), jnp.float32),
            pltpu.SemaphoreType.DMA,
            pltpu.SemaphoreType.DMA,
            pltpu.SemaphoreType.DMA,
        ],
        compiler_params=pltpu.CompilerParams(needs_layout_passes=False,
                                             use_tc_tiling_on_sc=False),
    )
    def sc_fn(edges, needed, utab, itab, zeros2d, neg1, out,
              map_ref, nbuf, ebufa, ebufb, cpk, sstage, tst0, tst1, rows_bf,
              rows, slotblk, cnt_ref, acc, sem, sem2, sem3):
        cid = lax.axis_index("c")
        sid = lax.axis_index("s")
        wid = cid * _NS + sid

        iota = lax.iota(jnp.int32, _L)

        zbase = pl.multiple_of(sid * _ZROWS, 8)
        pltpu.sync_copy(zeros2d.at[pl.ds(0, _ZROWS)],
                        acc.at[pl.ds(zbase, _ZROWS)])

        pltpu.sync_copy(neg1, map_ref)
        for c in range(_NSLOT // _NCH):
            pltpu.sync_copy(needed.at[pl.ds(c * _NCH, _NCH)], nbuf)

            def _map_body(i, carry, _c=c):
                for u in range(4):
                    off = pl.multiple_of(i * 64 + u * 16, 16)
                    vals = nbuf[pl.ds(off, _L)]
                    plsc.store_scatter(map_ref, [vals],
                                       iota + off + _c * _NCH)
                return carry

            lax.fori_loop(0, _NCH // 64, _map_body, 0)

        plsc.subcore_barrier()

        base_e = jnp.where(wid < _N_HI, wid * _T_HI * 128,
                           _N_HI * _T_HI * 128 + (wid - _N_HI) * _T_LO * 128)
        n_vec = jnp.where(wid < _N_HI, _VEC_HI, _VEC_LO)

        def _chunk_dma_base(k):
            chunk_lo = base_e + k * _CHUNK_E
            return pl.multiple_of(jnp.minimum(chunk_lo, _NE - _CHUNK_E), 128)

        def _fire_chunk(k):
            buf = ebufa if k % 2 == 0 else ebufb
            dmab = _chunk_dma_base(k)
            pltpu.async_copy(edges.at[:, pl.ds(dmab, _CHUNK_E)], buf, sem2)

        def _wait_chunk():
            pltpu.make_async_copy(edges.at[:, pl.ds(0, _CHUNK_E)],
                                  ebufa, sem2).wait()

        def _filter_vec(voff, cnts, buf):
            cnt_u, cnt_i = cnts
            off = pl.multiple_of(voff * _L, 16)
            d = buf[1, pl.ds(off, _L)]
            s = buf[0, pl.ds(off, _L)]
            slot = plsc.load_gather(map_ref, [d])
            m = slot >= 0
            is_u = s < _NU
            m_u = jnp.logical_and(m, is_u)
            m_i = jnp.logical_and(m, jnp.logical_not(is_u))
            sh = lax.shift_left(slot, 17)
            pos_u = cnt_u + plsc.cumsum(m_u.astype(jnp.int32)) - 1
            plsc.store_scatter(cpk, [pos_u], lax.bitwise_or(s, sh), mask=m_u)
            pos_i = (_CMAX - 1) - (cnt_i + plsc.cumsum(m_i.astype(jnp.int32))
                                   - 1)
            plsc.store_scatter(cpk, [pos_i],
                               lax.bitwise_or(s - _NU, sh), mask=m_i)
            return (cnt_u + plsc.all_reduce_population_count(m_u),
                    cnt_i + plsc.all_reduce_population_count(m_i))

        cnt = (jnp.zeros((_L,), jnp.int32), jnp.zeros((_L,), jnp.int32))
        _fire_chunk(0)
        for k in range(_N_CHUNKS):
            buf = ebufa if k % 2 == 0 else ebufb
            if k + 1 < _N_CHUNKS:
                _fire_chunk(k + 1)
            _wait_chunk()
            if k < _N_CHUNKS - 1:
                def _quad(i, c, _b=buf):
                    for u in range(4):
                        c = _filter_vec(i * 4 + u, c, _b)
                    return c
                cnt = lax.fori_loop(0, _CHUNK_V // 4, _quad, cnt)
            else:
                off_vec = lax.shift_right_logical(
                    base_e + k * _CHUNK_E - _chunk_dma_base(k), 4)
                nv = jnp.clip(n_vec - k * _CHUNK_V, 0, _CHUNK_V)

                def _one(i, c, _b=buf, _ov=off_vec):
                    return _filter_vec(_ov + i, c, _b)
                cnt = lax.fori_loop(0, nv, _one, cnt)

        cnt_ref[...] = cnt[0]
        n_u = cnt_ref[...][0]
        cnt_ref[...] = cnt[1]
        n_i = cnt_ref[...][0]
        npad_u = lax.bitwise_and(n_u + _BLK - 1, ~(_BLK - 1))
        npad_i = lax.bitwise_and(n_i + _BLK - 1, ~(_BLK - 1))
        dummy = jnp.full((_L,), _DUMMY << 17, jnp.int32)

        def _pad_u(v, carry):
            pos = iota + v * _L
            plsc.store_scatter(cpk, [pos], dummy, mask=pos >= n_u)
            return carry

        lax.fori_loop(lax.shift_right_logical(n_u, 4),
                      lax.shift_right_logical(npad_u, 4), _pad_u, 0)

        def _pad_i(v, carry):
            pos = iota + v * _L
            plsc.store_scatter(cpk, [pos], dummy, mask=pos < _CMAX - n_i)
            return carry

        lax.fori_loop(lax.shift_right_logical(_CMAX - npad_i, 4),
                      lax.shift_right_logical(_CMAX - n_i, 4)
                      + jnp.where(lax.bitwise_and(_CMAX - n_i, 15) > 0, 1, 0),
                      _pad_i, 0)

        nb_u = lax.shift_right_logical(npad_u, 6)
        nb = nb_u + lax.shift_right_logical(npad_i, 6)
        ibase = _CMAX - npad_i

        def _blk_off(j):
            return jnp.where(j < nb_u, j * _BLK,
                             ibase + (j - nb_u) * _BLK)

        def _fire_block(j, half):
            hbase = pl.multiple_of(half * _BLK, 8)
            boff = _blk_off(j)
            for v in range(4):
                off = pl.multiple_of(boff + v * 16, 16)
                w = cpk[pl.ds(off, _L)]
                sstage[pl.ds(pl.multiple_of(hbase + v * 16, 16), _L)] = \
                    lax.bitwise_and(w, (1 << 17) - 1)

            @pl.when(j < nb_u)
            def _():
                pltpu.async_copy(utab.at[sstage.at[pl.ds(hbase, _BLK)]],
                                 rows_bf.at[pl.ds(hbase, _BLK)], sem)

            @pl.when(j >= nb_u)
            def _():
                pltpu.async_copy(itab.at[sstage.at[pl.ds(hbase, _BLK)]],
                                 rows_bf.at[pl.ds(hbase, _BLK)], sem)

        @pl.when(nb > 0)
        def _():
            _fire_block(0, jnp.int32(0))

        tsts = [tst0, tst1]

        def _wait_scatter():
            pltpu.make_async_copy(rows.at[pl.ds(0, _BLK)],
                                  acc.at[tst0], sem3).wait()

        def _blk_body(j, carry):
            p = lax.bitwise_and(j, 1)

            @pl.when(j >= 1)
            def _():
                _wait_scatter()

            @pl.when(j + 1 < nb)
            def _():
                _fire_block(j + 1, 1 - p)

            pltpu.make_async_copy(utab.at[sstage.at[pl.ds(0, _BLK)]],
                                  rows_bf.at[pl.ds(0, _BLK)], sem).wait()
            boff = _blk_off(j)
            for q in range(2):
                @pl.when(p == q)
                def _(_q=q):
                    for v in range(4):
                        off = pl.multiple_of(boff + v * 16, 16)
                        w = cpk[pl.ds(off, _L)]
                        tsts[_q][pl.ds(v * 16, _L)] = \
                            lax.shift_right_logical(w, 17)

                    def _conv(r, carry2, _q=_q):
                        rr = _q * _BLK + r
                        for h in range(2):
                            wb = plsc.bitcast(
                                rows_bf[rr, pl.ds(h * 32, 32)], jnp.int32)
                            ev = plsc.bitcast(
                                lax.shift_left(wb, 16), jnp.float32)
                            od = plsc.bitcast(
                                lax.bitwise_and(
                                    wb, jnp.full((_L,), -65536, jnp.int32)),
                                jnp.float32)
                            rows[rr, pl.ds(h * 16, _L)] = ev
                            rows[rr, pl.ds(32 + h * 16, _L)] = od
                        return carry2

                    lax.fori_loop(0, _BLK, _conv, 0)
                    pltpu.async_copy(rows.at[pl.ds(_q * _BLK, _BLK)],
                                     acc.at[tsts[_q]], sem3, add=True)
            return carry

        lax.fori_loop(0, nb, _blk_body, 0)

        @pl.when(nb > 0)
        def _():
            _wait_scatter()

        plsc.subcore_barrier()

        jbase = pl.multiple_of(sid * _JPT, 16)
        pltpu.sync_copy(needed.at[pl.ds(jbase, _JPT)],
                        nbuf.at[pl.ds(0, _JPT)])
        for vb in range(_JPT // 128):
            for v in range(8):
                off = pl.multiple_of(vb * 128 + v * 16, 16)
                vals = nbuf[pl.ds(off, _L)]
                sl = plsc.load_gather(map_ref, [vals])
                slotblk[pl.ds(off, _L)] = sl
            pltpu.async_copy(acc.at[slotblk.at[pl.ds(vb * 128, 128)]],
                             rows.at[pl.ds(0, 128)], sem).wait()
            pltpu.sync_copy(rows.at[pl.ds(0, 128)],
                            out.at[cid, pl.ds(jbase + vb * 128, 128)])

    return sc_fn


def _tc_body(acc_ref, w_ref, b_ref, o_ref):
    a = acc_ref[0] + acc_ref[1]
    p = lax.dot_general(a, w_ref[...], (((1,), (1,)), ((), ())),
                        preferred_element_type=jnp.float32)
    p = p + b_ref[...]
    u = p[:_B]
    t = p[_B:]
    o_ref[...] = jnp.sum(u * t, axis=1, keepdims=True)


def kernel(user_indices, item_indices, edge_index, user_table, item_table,
           W, b):
    needed = jnp.concatenate([user_indices, item_indices + _NU])
    zeros2d = jnp.zeros((_ZROWS, _D), jnp.float32)
    neg1 = jnp.full((_NTOT,), -1, jnp.int32)

    sc_fn = _make_sc_kernel()
    partials = sc_fn(edge_index, needed,
                     user_table.astype(jnp.bfloat16),
                     item_table.astype(jnp.bfloat16), zeros2d, neg1)

    sigma = (list(range(0, 32, 2)) + list(range(32, 64, 2))
             + list(range(1, 32, 2)) + list(range(33, 64, 2)))
    Wp = W[:, jnp.array(sigma, dtype=jnp.int32)]
    out = pl.pallas_call(
        _tc_body,
        out_shape=jax.ShapeDtypeStruct((_B, 1), jnp.float32),
    )(partials, Wp, jnp.reshape(b, (1, _D)))
    return out


# speedup vs baseline: 1.2292x; 1.2292x over previous
import functools

import jax
import jax.numpy as jnp
from jax import lax
from jax.experimental import pallas as pl
from jax.experimental.pallas import tpu as pltpu
from jax.experimental.pallas import tpu_sc as plsc

_NU = 25000
_NTOT = 50000
_D = 64
_NE = 800000
_B = 4096
_NSLOT = 2 * _B
_L = 16
_NS = 16
_NC = 2
_NW = _NC * _NS

_NT128 = _NE // 128
_T_LO = _NT128 // _NW
_T_HI = _T_LO + 1
_N_HI = _NT128 - _NW * _T_LO
_VEC_HI = _T_HI * 8
_VEC_LO = _T_LO * 8
_CHUNK_V = 128
_CHUNK_E = _CHUNK_V * _L
_N_CHUNKS = (_VEC_HI + _CHUNK_V - 1) // _CHUNK_V
_CMAX = ((_VEC_HI * _L + 63) // 64) * 64
_BLK = 64
_DUMMY = _NSLOT
_ACC_ROWS = _NSLOT + 128
_ZROWS = _ACC_ROWS // _NS
_JPT = _NSLOT // _NS
_NCH = 2048


def _make_sc_kernel():
    mesh = plsc.VectorSubcoreMesh(core_axis_name="c", subcore_axis_name="s")

    @functools.partial(
        pl.kernel,
        out_type=jax.ShapeDtypeStruct((_NC, _NSLOT, _D), jnp.float32),
        mesh=mesh,
        scratch_types=[
            pltpu.VMEM((_NTOT,), jnp.int32),
            pltpu.VMEM((_NCH,), jnp.int32),
            pltpu.VMEM((2, _CHUNK_E), jnp.int32),
            pltpu.VMEM((2, _CHUNK_E), jnp.int32),
            pltpu.VMEM((_CMAX,), jnp.int32),
            pltpu.VMEM((2 * _BLK,), jnp.int32),
            pltpu.VMEM((_BLK,), jnp.int32),
            pltpu.VMEM((2 * _BLK, _D), jnp.float32),
            pltpu.VMEM((_JPT,), jnp.int32),
            pltpu.VMEM((_L,), jnp.int32),
            pltpu.VMEM_SHARED((_ACC_ROWS, _D), jnp.float32),
            pltpu.SemaphoreType.DMA,
            pltpu.SemaphoreType.DMA,
        ],
        compiler_params=pltpu.CompilerParams(needs_layout_passes=False,
                                             use_tc_tiling_on_sc=False),
    )
    def sc_fn(edges, needed, utab, itab, zeros2d, neg1, out,
              map_ref, nbuf, ebufa, ebufb, cpk, sstage, tstage, rows,
              slotblk, cnt_ref, acc, sem, sem2):
        cid = lax.axis_index("c")
        sid = lax.axis_index("s")
        wid = cid * _NS + sid

        iota = lax.iota(jnp.int32, _L)

        zbase = pl.multiple_of(sid * _ZROWS, 8)
        pltpu.sync_copy(zeros2d.at[pl.ds(0, _ZROWS)],
                        acc.at[pl.ds(zbase, _ZROWS)])

        pltpu.sync_copy(neg1, map_ref)
        for c in range(_NSLOT // _NCH):
            pltpu.sync_copy(needed.at[pl.ds(c * _NCH, _NCH)], nbuf)

            def _map_body(i, carry, _c=c):
                for u in range(4):
                    off = pl.multiple_of(i * 64 + u * 16, 16)
                    vals = nbuf[pl.ds(off, _L)]
                    plsc.store_scatter(map_ref, [vals],
                                       iota + off + _c * _NCH)
                return carry

            lax.fori_loop(0, _NCH // 64, _map_body, 0)

        plsc.subcore_barrier()

        base_e = jnp.where(wid < _N_HI, wid * _T_HI * 128,
                           _N_HI * _T_HI * 128 + (wid - _N_HI) * _T_LO * 128)
        n_vec = jnp.where(wid < _N_HI, _VEC_HI, _VEC_LO)

        def _chunk_dma_base(k):
            chunk_lo = base_e + k * _CHUNK_E
            return pl.multiple_of(jnp.minimum(chunk_lo, _NE - _CHUNK_E), 128)

        def _fire_chunk(k):
            buf = ebufa if k % 2 == 0 else ebufb
            dmab = _chunk_dma_base(k)
            pltpu.async_copy(edges.at[:, pl.ds(dmab, _CHUNK_E)], buf, sem2)

        def _wait_chunk():
            pltpu.make_async_copy(edges.at[:, pl.ds(0, _CHUNK_E)],
                                  ebufa, sem2).wait()

        def _filter_vec(voff, cnts, buf):
            cnt_u, cnt_i = cnts
            off = pl.multiple_of(voff * _L, 16)
            d = buf[1, pl.ds(off, _L)]
            s = buf[0, pl.ds(off, _L)]
            slot = plsc.load_gather(map_ref, [d])
            m = slot >= 0
            is_u = s < _NU
            m_u = jnp.logical_and(m, is_u)
            m_i = jnp.logical_and(m, jnp.logical_not(is_u))
            sh = lax.shift_left(slot, 17)
            pos_u = cnt_u + plsc.cumsum(m_u.astype(jnp.int32)) - 1
            plsc.store_scatter(cpk, [pos_u], lax.bitwise_or(s, sh), mask=m_u)
            pos_i = (_CMAX - 1) - (cnt_i + plsc.cumsum(m_i.astype(jnp.int32))
                                   - 1)
            plsc.store_scatter(cpk, [pos_i],
                               lax.bitwise_or(s - _NU, sh), mask=m_i)
            return (cnt_u + plsc.all_reduce_population_count(m_u),
                    cnt_i + plsc.all_reduce_population_count(m_i))

        cnt = (jnp.zeros((_L,), jnp.int32), jnp.zeros((_L,), jnp.int32))
        _fire_chunk(0)
        for k in range(_N_CHUNKS):
            buf = ebufa if k % 2 == 0 else ebufb
            if k + 1 < _N_CHUNKS:
                _fire_chunk(k + 1)
            _wait_chunk()
            if k < _N_CHUNKS - 1:
                def _quad(i, c, _b=buf):
                    for u in range(4):
                        c = _filter_vec(i * 4 + u, c, _b)
                    return c
                cnt = lax.fori_loop(0, _CHUNK_V // 4, _quad, cnt)
            else:
                off_vec = lax.shift_right_logical(
                    base_e + k * _CHUNK_E - _chunk_dma_base(k), 4)
                nv = jnp.clip(n_vec - k * _CHUNK_V, 0, _CHUNK_V)

                def _one(i, c, _b=buf, _ov=off_vec):
                    return _filter_vec(_ov + i, c, _b)
                cnt = lax.fori_loop(0, nv, _one, cnt)

        cnt_ref[...] = cnt[0]
        n_u = cnt_ref[...][0]
        cnt_ref[...] = cnt[1]
        n_i = cnt_ref[...][0]
        npad_u = lax.bitwise_and(n_u + _BLK - 1, ~(_BLK - 1))
        npad_i = lax.bitwise_and(n_i + _BLK - 1, ~(_BLK - 1))
        dummy = jnp.full((_L,), _DUMMY << 17, jnp.int32)

        def _pad_u(v, carry):
            pos = iota + v * _L
            plsc.store_scatter(cpk, [pos], dummy, mask=pos >= n_u)
            return carry

        lax.fori_loop(lax.shift_right_logical(n_u, 4),
                      lax.shift_right_logical(npad_u, 4), _pad_u, 0)

        def _pad_i(v, carry):
            pos = iota + v * _L
            plsc.store_scatter(cpk, [pos], dummy, mask=pos < _CMAX - n_i)
            return carry

        lax.fori_loop(lax.shift_right_logical(_CMAX - npad_i, 4),
                      lax.shift_right_logical(_CMAX - n_i, 4)
                      + jnp.where(lax.bitwise_and(_CMAX - n_i, 15) > 0, 1, 0),
                      _pad_i, 0)

        nb_u = lax.shift_right_logical(npad_u, 6)
        nb = nb_u + lax.shift_right_logical(npad_i, 6)
        ibase = _CMAX - npad_i

        def _blk_off(j):
            return jnp.where(j < nb_u, j * _BLK,
                             ibase + (j - nb_u) * _BLK)

        def _fire_block(j, half):
            hbase = pl.multiple_of(half * _BLK, 8)
            boff = _blk_off(j)
            for v in range(4):
                off = pl.multiple_of(boff + v * 16, 16)
                w = cpk[pl.ds(off, _L)]
                sstage[pl.ds(pl.multiple_of(hbase + v * 16, 16), _L)] = \
                    lax.bitwise_and(w, (1 << 17) - 1)

            @pl.when(j < nb_u)
            def _():
                pltpu.async_copy(utab.at[sstage.at[pl.ds(hbase, _BLK)]],
                                 rows.at[pl.ds(hbase, _BLK)], sem)

            @pl.when(j >= nb_u)
            def _():
                pltpu.async_copy(itab.at[sstage.at[pl.ds(hbase, _BLK)]],
                                 rows.at[pl.ds(hbase, _BLK)], sem)

        @pl.when(nb > 0)
        def _():
            _fire_block(0, jnp.int32(0))

        def _blk_body(j, carry):
            p = lax.bitwise_and(j, 1)

            @pl.when(j + 1 < nb)
            def _():
                _fire_block(j + 1, 1 - p)

            pltpu.make_async_copy(utab.at[sstage.at[pl.ds(0, _BLK)]],
                                  rows.at[pl.ds(0, _BLK)], sem).wait()
            boff = _blk_off(j)
            for v in range(4):
                off = pl.multiple_of(boff + v * 16, 16)
                w = cpk[pl.ds(off, _L)]
                tstage[pl.ds(v * 16, _L)] = lax.shift_right_logical(w, 17)
            pbase = pl.multiple_of(p * _BLK, 8)
            pltpu.sync_copy(rows.at[pl.ds(pbase, _BLK)], acc.at[tstage],
                            add=True)
            return carry

        lax.fori_loop(0, nb, _blk_body, 0)

        plsc.subcore_barrier()

        jbase = pl.multiple_of(sid * _JPT, 16)
        pltpu.sync_copy(needed.at[pl.ds(jbase, _JPT)],
                        nbuf.at[pl.ds(0, _JPT)])
        for vb in range(_JPT // 128):
            for v in range(8):
                off = pl.multiple_of(vb * 128 + v * 16, 16)
                vals = nbuf[pl.ds(off, _L)]
                sl = plsc.load_gather(map_ref, [vals])
                slotblk[pl.ds(off, _L)] = sl
            pltpu.async_copy(acc.at[slotblk.at[pl.ds(vb * 128, 128)]],
                             rows.at[pl.ds(0, 128)], sem).wait()
            pltpu.sync_copy(rows.at[pl.ds(0, 128)],
                            out.at[cid, pl.ds(jbase + vb * 128, 128)])

    return sc_fn


def _tc_body(acc_ref, w_ref, b_ref, o_ref):
    a = acc_ref[0] + acc_ref[1]
    p = lax.dot_general(a, w_ref[...], (((1,), (1,)), ((), ())),
                        preferred_element_type=jnp.float32)
    p = p + b_ref[...]
    u = p[:_B]
    t = p[_B:]
    o_ref[...] = jnp.sum(u * t, axis=1, keepdims=True)


def kernel(user_indices, item_indices, edge_index, user_table, item_table,
           W, b):
    needed = jnp.concatenate([user_indices, item_indices + _NU])
    zeros2d = jnp.zeros((_ZROWS, _D), jnp.float32)
    neg1 = jnp.full((_NTOT,), -1, jnp.int32)

    sc_fn = _make_sc_kernel()
    partials = sc_fn(edge_index, needed, user_table, item_table, zeros2d,
                     neg1)

    out = pl.pallas_call(
        _tc_body,
        out_shape=jax.ShapeDtypeStruct((_B, 1), jnp.float32),
    )(partials, W, jnp.reshape(b, (1, _D)))
    return out
